# Initial kernel scaffold; baseline (speedup 1.0000x reference)
import jax
import jax.numpy as jnp
from jax.experimental import pallas as pl


def kernel(input_ids, attention_mask, position_ids, labels, embedding_table):
    raise NotImplementedError("write your pallas kernel here")


# speedup vs baseline: 1.3845x; 1.3845x over previous
import functools
import jax
import jax.numpy as jnp
from jax import lax
from jax.experimental import pallas as pl
from jax.experimental.pallas import tpu as pltpu
from jax.experimental.pallas import tpu_sc as plsc

VOCAB = 100000
D = 768
B = 4
S = 2048
N = B * S

NC, NS = 2, 16
NW = NC * NS
PER_W = N // NW
CHUNK = 64
NCHUNK = PER_W // CHUNK


def _gather_body(idx_hbm, table_hbm, out_hbm, idx_v, rows_v, sem):
    wid = lax.axis_index("s") * NC + lax.axis_index("c")
    base = wid * PER_W
    pltpu.sync_copy(idx_hbm.at[wid], idx_v)
    for j in range(NCHUNK):
        pltpu.async_copy(table_hbm.at[idx_v.at[j]], rows_v, sem).wait()
        pltpu.sync_copy(rows_v, out_hbm.at[pl.ds(base + j * CHUNK, CHUNK)])


@jax.jit
def _embed_lookup(input_ids, embedding_table):
    idx = input_ids.reshape(NW, NCHUNK, CHUNK)
    mesh = plsc.VectorSubcoreMesh(core_axis_name="c", subcore_axis_name="s")
    k = pl.kernel(
        _gather_body,
        out_type=jax.ShapeDtypeStruct((N, D), jnp.float32),
        mesh=mesh,
        scratch_types=[
            pltpu.VMEM((NCHUNK, CHUNK), jnp.int32),
            pltpu.VMEM((CHUNK, D), jnp.float32),
            pltpu.SemaphoreType.DMA,
        ],
    )
    out = k(idx, embedding_table)
    return out.reshape(B, S, D)


def kernel(input_ids, attention_mask, position_ids, labels, embedding_table):
    hidden_states = _embed_lookup(input_ids, embedding_table)
    return (hidden_states, attention_mask, position_ids, labels)


# speedup vs baseline: 1.4503x; 1.0475x over previous
import functools
import jax
import jax.numpy as jnp
from jax import lax
from jax.experimental import pallas as pl
from jax.experimental.pallas import tpu as pltpu
from jax.experimental.pallas import tpu_sc as plsc

VOCAB = 100000
D = 768
B = 4
S = 2048
N = B * S

NC, NS = 2, 16
NW = NC * NS
PER_W = N // NW
CHUNK = 64
NCHUNK = PER_W // CHUNK


def _gather_body(idx_hbm, table_hbm, out_hbm, idx_v, rows_a, rows_b,
                 gsem_a, gsem_b, ssem_a, ssem_b):
    wid = lax.axis_index("s") * NC + lax.axis_index("c")
    base = wid * PER_W
    pltpu.sync_copy(idx_hbm.at[wid], idx_v)

    rows = (rows_a, rows_b)
    gsem = (gsem_a, gsem_b)
    ssem = (ssem_a, ssem_b)

    gathers = [None, None]
    scatters = [None, None]
    for j in range(NCHUNK):
        b = j % 2
        if scatters[b] is not None:
            scatters[b].wait()
        gathers[b] = pltpu.async_copy(table_hbm.at[idx_v.at[j]], rows[b],
                                      gsem[b])
        p = 1 - b
        if gathers[p] is not None:
            gathers[p].wait()
            scatters[p] = pltpu.async_copy(
                rows[p], out_hbm.at[pl.ds(base + (j - 1) * CHUNK, CHUNK)],
                ssem[p])
    last = (NCHUNK - 1) % 2
    gathers[last].wait()
    scatters[last] = pltpu.async_copy(
        rows[last], out_hbm.at[pl.ds(base + (NCHUNK - 1) * CHUNK, CHUNK)],
        ssem[last])
    scatters[1 - last].wait()
    scatters[last].wait()


@jax.jit
def _embed_lookup(input_ids, embedding_table):
    idx = input_ids.reshape(NW, NCHUNK, CHUNK)
    mesh = plsc.VectorSubcoreMesh(core_axis_name="c", subcore_axis_name="s")
    k = pl.kernel(
        _gather_body,
        out_type=jax.ShapeDtypeStruct((N, D), jnp.float32),
        mesh=mesh,
        scratch_types=[
            pltpu.VMEM((NCHUNK, CHUNK), jnp.int32),
            pltpu.VMEM((CHUNK, D), jnp.float32),
            pltpu.VMEM((CHUNK, D), jnp.float32),
            pltpu.SemaphoreType.DMA,
            pltpu.SemaphoreType.DMA,
            pltpu.SemaphoreType.DMA,
            pltpu.SemaphoreType.DMA,
        ],
    )
    out = k(idx, embedding_table)
    return out.reshape(B, S, D)


def kernel(input_ids, attention_mask, position_ids, labels, embedding_table):
    hidden_states = _embed_lookup(input_ids, embedding_table)
    return (hidden_states, attention_mask, position_ids, labels)


# speedup vs baseline: 1.4888x; 1.0265x over previous
import jax
import jax.numpy as jnp
from jax import lax
from jax.experimental import pallas as pl
from jax.experimental.pallas import tpu as pltpu
from jax.experimental.pallas import tpu_sc as plsc

VOCAB = 100000
D = 768
B = 4
S = 2048
N = B * S

NC, NS = 2, 16
NW = NC * NS
PER_W = N // NW
W_PER_B = S // PER_W
CHUNK = 32
NCHUNK = PER_W // CHUNK
NBUF = 4


def _gather_body(ids_hbm, mask_hbm, pos_hbm, lab_hbm, table_hbm,
                 out_hbm, omask_hbm, opos_hbm, olab_hbm,
                 idx_v, rows0, rows1, rows2, rows3,
                 gs0, gs1, gs2, gs3, ss0, ss1, ss2, ss3, psem):
    wid = lax.axis_index("s") * NC + lax.axis_index("c")
    b = wid // W_PER_B
    col = (wid % W_PER_B) * PER_W

    p0 = pltpu.async_copy(mask_hbm.at[b, pl.ds(col, PER_W)],
                          omask_hbm.at[b, pl.ds(col, PER_W)], psem)
    p1 = pltpu.async_copy(pos_hbm.at[b, pl.ds(col, PER_W)],
                          opos_hbm.at[b, pl.ds(col, PER_W)], psem)
    p2 = pltpu.async_copy(lab_hbm.at[b, pl.ds(col, PER_W)],
                          olab_hbm.at[b, pl.ds(col, PER_W)], psem)

    pltpu.sync_copy(ids_hbm.at[b, pl.ds(col, PER_W)], idx_v)

    rows = (rows0, rows1, rows2, rows3)
    gsem = (gs0, gs1, gs2, gs3)
    ssem = (ss0, ss1, ss2, ss3)

    gathers = [None] * NBUF
    scatters = [None] * NBUF
    for j in range(NCHUNK):
        r = j % NBUF
        if scatters[r] is not None:
            scatters[r].wait()
        gathers[r] = pltpu.async_copy(
            table_hbm.at[idx_v.at[pl.ds(j * CHUNK, CHUNK)]], rows[r],
            gsem[r])
        if j > 0:
            p = (j - 1) % NBUF
            gathers[p].wait()
            scatters[p] = pltpu.async_copy(
                rows[p], out_hbm.at[b, pl.ds(col + (j - 1) * CHUNK, CHUNK)],
                ssem[p])
    last = (NCHUNK - 1) % NBUF
    gathers[last].wait()
    scatters[last] = pltpu.async_copy(
        rows[last], out_hbm.at[b, pl.ds(col + (NCHUNK - 1) * CHUNK, CHUNK)],
        ssem[last])
    for j in range(max(0, NCHUNK - NBUF + 1), NCHUNK):
        scatters[j % NBUF].wait()
    p0.wait()
    p1.wait()
    p2.wait()


@jax.jit
def _embed_lookup(input_ids, attention_mask, position_ids, labels,
                  embedding_table):
    mesh = plsc.VectorSubcoreMesh(core_axis_name="c", subcore_axis_name="s")
    k = pl.kernel(
        _gather_body,
        out_type=(
            jax.ShapeDtypeStruct((B, S, D), jnp.float32),
            jax.ShapeDtypeStruct((B, S), jnp.int32),
            jax.ShapeDtypeStruct((B, S), jnp.int32),
            jax.ShapeDtypeStruct((B, S), jnp.int32),
        ),
        mesh=mesh,
        scratch_types=(
            [pltpu.VMEM((PER_W,), jnp.int32)]
            + [pltpu.VMEM((CHUNK, D), jnp.float32) for _ in range(NBUF)]
            + [pltpu.SemaphoreType.DMA] * (2 * NBUF + 1)
        ),
    )
    return k(input_ids, attention_mask, position_ids, labels,
             embedding_table)


def kernel(input_ids, attention_mask, position_ids, labels, embedding_table):
    return _embed_lookup(input_ids, attention_mask, position_ids, labels,
                         embedding_table)


# speedup vs baseline: 1.5266x; 1.0254x over previous
import jax
import jax.numpy as jnp
from jax import lax
from jax.experimental import pallas as pl
from jax.experimental.pallas import tpu as pltpu
from jax.experimental.pallas import tpu_sc as plsc

VOCAB = 100000
D = 768
B = 4
S = 2048
N = B * S

NC, NS = 2, 16
NW = NC * NS
PER_W = N // NW
W_PER_B = S // PER_W
CHUNK = 64
NCHUNK = PER_W // CHUNK
NBUF = 2


def _gather_body(ids_hbm, mask_hbm, pos_hbm, lab_hbm, table_hbm,
                 out_hbm, omask_hbm, opos_hbm, olab_hbm,
                 idx_v, *bufs_and_sems):
    rows = bufs_and_sems[:NBUF]
    gsem = bufs_and_sems[NBUF:2 * NBUF]
    ssem = bufs_and_sems[2 * NBUF:3 * NBUF]
    psem = bufs_and_sems[3 * NBUF]

    wid = lax.axis_index("s") * NC + lax.axis_index("c")
    b = wid // W_PER_B
    col = (wid % W_PER_B) * PER_W

    p0 = pltpu.async_copy(mask_hbm.at[b, pl.ds(col, PER_W)],
                          omask_hbm.at[b, pl.ds(col, PER_W)], psem)
    p1 = pltpu.async_copy(pos_hbm.at[b, pl.ds(col, PER_W)],
                          opos_hbm.at[b, pl.ds(col, PER_W)], psem)
    p2 = pltpu.async_copy(lab_hbm.at[b, pl.ds(col, PER_W)],
                          olab_hbm.at[b, pl.ds(col, PER_W)], psem)

    pltpu.sync_copy(ids_hbm.at[b, pl.ds(col, PER_W)], idx_v)

    gathers = [None] * NBUF
    scatters = [None] * NBUF
    for j in range(NCHUNK):
        r = j % NBUF
        if scatters[r] is not None:
            scatters[r].wait()
        gathers[r] = pltpu.async_copy(
            table_hbm.at[idx_v.at[pl.ds(j * CHUNK, CHUNK)]], rows[r],
            gsem[r])
        if j > 0:
            p = (j - 1) % NBUF
            gathers[p].wait()
            scatters[p] = pltpu.async_copy(
                rows[p], out_hbm.at[b, pl.ds(col + (j - 1) * CHUNK, CHUNK)],
                ssem[p])
    last = (NCHUNK - 1) % NBUF
    gathers[last].wait()
    scatters[last] = pltpu.async_copy(
        rows[last], out_hbm.at[b, pl.ds(col + (NCHUNK - 1) * CHUNK, CHUNK)],
        ssem[last])
    for j in range(max(0, NCHUNK - NBUF + 1), NCHUNK):
        scatters[j % NBUF].wait()
    p0.wait()
    p1.wait()
    p2.wait()


@jax.jit
def _embed_lookup(input_ids, attention_mask, position_ids, labels,
                  embedding_table):
    mesh = plsc.VectorSubcoreMesh(core_axis_name="c", subcore_axis_name="s")
    k = pl.kernel(
        _gather_body,
        out_type=(
            jax.ShapeDtypeStruct((B, S, D), jnp.float32),
            jax.ShapeDtypeStruct((B, S), jnp.int32),
            jax.ShapeDtypeStruct((B, S), jnp.int32),
            jax.ShapeDtypeStruct((B, S), jnp.int32),
        ),
        mesh=mesh,
        scratch_types=(
            [pltpu.VMEM((PER_W,), jnp.int32)]
            + [pltpu.VMEM((CHUNK, D), jnp.float32) for _ in range(NBUF)]
            + [pltpu.SemaphoreType.DMA] * (2 * NBUF + 1)
        ),
    )
    return k(input_ids, attention_mask, position_ids, labels,
             embedding_table)


def kernel(input_ids, attention_mask, position_ids, labels, embedding_table):
    return _embed_lookup(input_ids, attention_mask, position_ids, labels,
                         embedding_table)
